# baseline (speedup 1.0000x reference)
import functools

import jax
import jax.numpy as jnp
from jax import lax
from jax.experimental import pallas as pl
from jax.experimental.pallas import tpu as pltpu
from jax.experimental.pallas import tpu_sc as plsc

N = 100000
C = 256
G = 64
H = 16

NC = 2
NS = 16
NW = NC * NS
CH = 128
NCHT = (N + CH - 1) // CH
SACR = G
ACC_ROWS = 72

BLK2 = 10000
NBLK2 = N // BLK2

_sc_mesh = plsc.VectorSubcoreMesh(core_axis_name="c", subcore_axis_name="s")


AROW = 144


@functools.partial(
    pl.kernel,
    out_type=jax.ShapeDtypeStruct((NW, AROW, C), jnp.float32),
    mesh=_sc_mesh,
    scratch_types=[
        pltpu.VMEM((CH, C), jnp.float32),
        pltpu.VMEM((CH, C), jnp.float32),
        pltpu.VMEM((CH + 16,), jnp.int32),
        pltpu.VMEM((CH + 16,), jnp.int32),
        pltpu.VMEM((AROW, C), jnp.float32),
        pltpu.SemaphoreType.DMA,
        pltpu.SemaphoreType.DMA,
        pltpu.SemaphoreType.DMA,
        pltpu.SemaphoreType.DMA,
    ],
)
def _sc_pool(x_hbm, bidx_hbm, out_hbm, xbuf0, xbuf1, idbuf0, idbuf1,
             acc, sx0, sx1, si0, si1):
    core = lax.axis_index("c")
    sub = lax.axis_index("s")
    wid = core * NS + sub

    nch_w = 24 + (wid < 14).astype(jnp.int32)
    start = wid * 24 + jnp.minimum(wid, 14)

    zero16 = jnp.zeros((16,), jnp.float32)

    def _zero(r, _):
        for j in range(16):
            acc[r, pl.ds(j * 16, 16)] = zero16
        return 0
    lax.fori_loop(0, AROW, _zero, 0)

    def _x_desc(c, buf, sem):
        off = jnp.minimum(c * CH, N - CH)
        return pltpu.make_async_copy(x_hbm.at[pl.ds(off, CH)], buf, sem)

    def _id_desc(c, buf, sem):
        return pltpu.make_async_copy(bidx_hbm.at[pl.ds(c * CH, CH)],
                                     buf.at[pl.ds(0, CH)], sem)

    def _issue(c, xbuf, idbuf, sx, si):
        _x_desc(c, xbuf, sx).start()
        _id_desc(c, idbuf, si).start()

    _issue(start, xbuf0, idbuf0, sx0, si0)
    _issue(start + 1, xbuf1, idbuf1, sx1, si1)

    def _flush(seg, racc, n):
        for j in range(16):
            acc[seg, pl.ds(j * 16, 16)] = acc[seg, pl.ds(j * 16, 16)] + racc[j]
        cr = seg + 72
        acc[cr, pl.ds(0, 16)] = (acc[cr, pl.ds(0, 16)]
                                 + jnp.full((16,), 1.0, jnp.float32)
                                 * n.astype(jnp.float32))

    def _step(k, xbuf, idbuf, sx, si):
        c = start + k
        _x_desc(c, xbuf, sx).wait()
        _id_desc(c, idbuf, si).wait()

        def _row(r, carry):
            prev, n = carry[0], carry[1]
            racc = carry[2:]
            seg = idbuf[pl.ds(r, 16)][0]
            neq = seg != prev

            @pl.when(neq)
            def _():
                _flush(prev, racc, n)

            nracc = tuple(
                jnp.where(neq, zero16, racc[j]) + xbuf[r, pl.ds(j * 16, 16)]
                for j in range(16))
            nn = jnp.where(neq, 1, n + 1)
            return (seg, nn) + nracc

        init = (idbuf[pl.ds(0, 16)][0], jnp.int32(0)) + (zero16,) * 16
        last = lax.fori_loop(0, CH, _row, init)
        _flush(last[0], last[2:], last[1])

        @pl.when(k + 2 < nch_w)
        def _():
            _issue(c + 2, xbuf, idbuf, sx, si)

    def _body(k, _):
        @pl.when(k < nch_w)
        def _():
            @pl.when(k % 2 == 0)
            def _():
                _step(k, xbuf0, idbuf0, sx0, si0)

            @pl.when(k % 2 == 1)
            def _():
                _step(k, xbuf1, idbuf1, sx1, si1)
        return 0

    lax.fori_loop(0, 25, _body, 0)

    pltpu.sync_copy(acc, out_hbm.at[wid])


def _scale_kernel(x_ref, b_ref, part_ref, w1_ref, w2_ref, out_ref, scale_ref):
    i = pl.program_id(0)

    @pl.when(i == 0)
    def _mlp():
        s = jnp.sum(part_ref[:, 0:G, :], axis=0)
        cnt = jnp.sum(part_ref[:, 72:72 + G, 0:1], axis=0)
        mean = s / jnp.maximum(cnt, 1.0)
        h = jax.lax.dot_general(mean, w1_ref[...], (((1,), (1,)), ((), ())),
                                preferred_element_type=jnp.float32)
        h = jnp.maximum(h, 0.0)
        logits = jax.lax.dot_general(h, w2_ref[...], (((1,), (1,)), ((), ())),
                                     preferred_element_type=jnp.float32)
        scale_ref[...] = jax.nn.sigmoid(logits)

    seg = b_ref[0, 0, :]
    gids = lax.broadcasted_iota(jnp.int32, (BLK2, G), 1)
    onehot = (gids == seg[:, None]).astype(jnp.float32)
    rows = jax.lax.dot_general(onehot, scale_ref[...], (((1,), (0,)), ((), ())),
                               preferred_element_type=jnp.float32)
    out_ref[...] = x_ref[...] * rows


def kernel(x, batch, W1, W2):
    b32 = batch.astype(jnp.int32)
    n_tail = N - (NCHT - 1) * CH
    bidx = jnp.concatenate([
        b32[:(NCHT - 1) * CH],
        jnp.full((CH - n_tail,), SACR, jnp.int32),
        b32[(NCHT - 1) * CH:],
    ])

    part = _sc_pool(x, bidx)

    b2 = b32.reshape(NBLK2, 1, BLK2)
    out = pl.pallas_call(
        _scale_kernel,
        grid=(NBLK2,),
        in_specs=[
            pl.BlockSpec((BLK2, C), lambda i: (i, 0)),
            pl.BlockSpec((1, 1, BLK2), lambda i: (i, 0, 0)),
            pl.BlockSpec((NW, AROW, C), lambda i: (0, 0, 0)),
            pl.BlockSpec((H, C), lambda i: (0, 0)),
            pl.BlockSpec((C, H), lambda i: (0, 0)),
        ],
        out_specs=pl.BlockSpec((BLK2, C), lambda i: (i, 0)),
        out_shape=jax.ShapeDtypeStruct((N, C), jnp.float32),
        scratch_shapes=[pltpu.VMEM((G, C), jnp.float32)],
    )(x, b2, part, W1, W2)
    return out


# speedup vs baseline: 1.2817x; 1.2817x over previous
import functools

import jax
import jax.numpy as jnp
from jax import lax
from jax.experimental import pallas as pl
from jax.experimental.pallas import tpu as pltpu
from jax.experimental.pallas import tpu_sc as plsc

N = 100000
C = 256
G = 64
H = 16

NC = 2
NS = 16
NW = NC * NS
CH = 128
NCHT = (N + CH - 1) // CH
SACR = G
ACC_ROWS = 72

BLK2 = 10000
NBLK2 = N // BLK2

_sc_mesh = plsc.VectorSubcoreMesh(core_axis_name="c", subcore_axis_name="s")


AROW = 144


@functools.partial(
    pl.kernel,
    out_type=jax.ShapeDtypeStruct((NW, AROW, C), jnp.float32),
    mesh=_sc_mesh,
    scratch_types=[
        pltpu.VMEM((CH, C), jnp.float32),
        pltpu.VMEM((CH, C), jnp.float32),
        pltpu.VMEM((CH + 16,), jnp.int32),
        pltpu.VMEM((CH + 16,), jnp.int32),
        pltpu.VMEM((AROW, C), jnp.float32),
        pltpu.VMEM((8, C), jnp.float32),
        pltpu.SMEM((2,), jnp.int32),
        pltpu.SemaphoreType.DMA,
        pltpu.SemaphoreType.DMA,
        pltpu.SemaphoreType.DMA,
        pltpu.SemaphoreType.DMA,
    ],
)
def _sc_pool(x_hbm, bidx_hbm, out_hbm, xbuf0, xbuf1, idbuf0, idbuf1,
             acc, racc, sreg, sx0, sx1, si0, si1):
    core = lax.axis_index("c")
    sub = lax.axis_index("s")
    wid = core * NS + sub

    nch_w = 24 + (wid < 14).astype(jnp.int32)
    start = wid * 24 + jnp.minimum(wid, 14)

    zero16 = jnp.zeros((16,), jnp.float32)

    def _zero(r, _):
        for j in range(16):
            acc[r, pl.ds(j * 16, 16)] = zero16
        return 0
    lax.fori_loop(0, AROW, _zero, 0)

    def _x_desc(c, buf, sem):
        off = jnp.minimum(c * CH, N - CH)
        return pltpu.make_async_copy(x_hbm.at[pl.ds(off, CH)], buf, sem)

    def _id_desc(c, buf, sem):
        return pltpu.make_async_copy(bidx_hbm.at[pl.ds(c * CH, CH)],
                                     buf.at[pl.ds(0, CH)], sem)

    def _issue(c, xbuf, idbuf, sx, si):
        _x_desc(c, xbuf, sx).start()
        _id_desc(c, idbuf, si).start()

    _issue(start, xbuf0, idbuf0, sx0, si0)
    _issue(start + 1, xbuf1, idbuf1, sx1, si1)

    def _zr(j, _):
        racc[0, pl.ds(j * 16, 16)] = zero16
        return 0
    lax.fori_loop(0, 16, _zr, 0)
    sreg[1] = 0
    sreg[0] = jnp.int32(SACR)

    def _flush():
        prev = sreg[0]
        for j in range(16):
            acc[prev, pl.ds(j * 16, 16)] = (acc[prev, pl.ds(j * 16, 16)]
                                            + racc[0, pl.ds(j * 16, 16)])
            racc[0, pl.ds(j * 16, 16)] = zero16
        cr = prev + 72
        acc[cr, pl.ds(0, 16)] = (acc[cr, pl.ds(0, 16)]
                                 + jnp.full((16,), 1.0, jnp.float32)
                                 * sreg[1].astype(jnp.float32))
        sreg[1] = 0

    def _step(k, xbuf, idbuf, sx, si):
        c = start + k
        _x_desc(c, xbuf, sx).wait()
        _id_desc(c, idbuf, si).wait()

        def _group(g, _):
            r0 = g * 16
            idv = idbuf[pl.ds(r0, 16)]
            first = idv[0]
            uniform = first == idv[15]
            prev = sreg[0]

            @pl.when(uniform & (first == prev))
            def _():
                for j in range(16):
                    sl = pl.ds(j * 16, 16)
                    v = (((xbuf[r0 + 0, sl] + xbuf[r0 + 1, sl])
                          + (xbuf[r0 + 2, sl] + xbuf[r0 + 3, sl]))
                         + ((xbuf[r0 + 4, sl] + xbuf[r0 + 5, sl])
                            + (xbuf[r0 + 6, sl] + xbuf[r0 + 7, sl])))
                    w = (((xbuf[r0 + 8, sl] + xbuf[r0 + 9, sl])
                          + (xbuf[r0 + 10, sl] + xbuf[r0 + 11, sl]))
                         + ((xbuf[r0 + 12, sl] + xbuf[r0 + 13, sl])
                            + (xbuf[r0 + 14, sl] + xbuf[r0 + 15, sl])))
                    racc[0, sl] = racc[0, sl] + (v + w)
                sreg[1] = sreg[1] + 16

            @pl.when(jnp.logical_not(uniform & (first == prev)))
            def _():
                def _row(r, _):
                    seg = idbuf[pl.ds(r, 16)][0]

                    @pl.when(seg != sreg[0])
                    def _():
                        _flush()
                        sreg[0] = seg

                    for j in range(16):
                        sl = pl.ds(j * 16, 16)
                        racc[0, sl] = racc[0, sl] + xbuf[r, sl]
                    sreg[1] = sreg[1] + 1
                    return 0

                lax.fori_loop(r0, r0 + 16, _row, 0)
            return 0

        lax.fori_loop(0, CH // 16, _group, 0)

        @pl.when(k + 2 < nch_w)
        def _():
            _issue(c + 2, xbuf, idbuf, sx, si)

    def _body(k, _):
        @pl.when(k < nch_w)
        def _():
            @pl.when(k % 2 == 0)
            def _():
                _step(k, xbuf0, idbuf0, sx0, si0)

            @pl.when(k % 2 == 1)
            def _():
                _step(k, xbuf1, idbuf1, sx1, si1)
        return 0

    lax.fori_loop(0, 25, _body, 0)
    _flush()

    pltpu.sync_copy(acc, out_hbm.at[wid])


def _scale_kernel(x_ref, b_ref, part_ref, w1_ref, w2_ref, out_ref, scale_ref):
    i = pl.program_id(0)

    @pl.when(i == 0)
    def _mlp():
        s = jnp.sum(part_ref[:, 0:G, :], axis=0)
        cnt = jnp.sum(part_ref[:, 72:72 + G, 0:1], axis=0)
        mean = s / jnp.maximum(cnt, 1.0)
        h = jax.lax.dot_general(mean, w1_ref[...], (((1,), (1,)), ((), ())),
                                preferred_element_type=jnp.float32)
        h = jnp.maximum(h, 0.0)
        logits = jax.lax.dot_general(h, w2_ref[...], (((1,), (1,)), ((), ())),
                                     preferred_element_type=jnp.float32)
        scale_ref[...] = jax.nn.sigmoid(logits)

    seg = b_ref[0, 0, :]
    gids = lax.broadcasted_iota(jnp.int32, (BLK2, G), 1)
    onehot = (gids == seg[:, None]).astype(jnp.float32)
    rows = jax.lax.dot_general(onehot, scale_ref[...], (((1,), (0,)), ((), ())),
                               preferred_element_type=jnp.float32)
    out_ref[...] = x_ref[...] * rows


def kernel(x, batch, W1, W2):
    b32 = batch.astype(jnp.int32)
    n_tail = N - (NCHT - 1) * CH
    bidx = jnp.concatenate([
        b32[:(NCHT - 1) * CH],
        jnp.full((CH - n_tail,), SACR, jnp.int32),
        b32[(NCHT - 1) * CH:],
    ])

    part = _sc_pool(x, bidx)

    b2 = b32.reshape(NBLK2, 1, BLK2)
    out = pl.pallas_call(
        _scale_kernel,
        grid=(NBLK2,),
        in_specs=[
            pl.BlockSpec((BLK2, C), lambda i: (i, 0)),
            pl.BlockSpec((1, 1, BLK2), lambda i: (i, 0, 0)),
            pl.BlockSpec((NW, AROW, C), lambda i: (0, 0, 0)),
            pl.BlockSpec((H, C), lambda i: (0, 0)),
            pl.BlockSpec((C, H), lambda i: (0, 0)),
        ],
        out_specs=pl.BlockSpec((BLK2, C), lambda i: (i, 0)),
        out_shape=jax.ShapeDtypeStruct((N, C), jnp.float32),
        scratch_shapes=[pltpu.VMEM((G, C), jnp.float32)],
    )(x, b2, part, W1, W2)
    return out
